# Initial kernel scaffold; baseline (speedup 1.0000x reference)
import jax
import jax.numpy as jnp
from jax.experimental import pallas as pl


def kernel(output, perturbation, y):
    raise NotImplementedError("write your pallas kernel here")


# speedup vs baseline: 1.4797x; 1.4797x over previous
import functools

import jax
import jax.numpy as jnp
from jax import lax
from jax.experimental import pallas as pl
from jax.experimental.pallas import tpu as pltpu

_MU = 0.5


def _loss_kernel(out_ref, pert_ref, y_ref, loss_ref, *, mu):
    logits = out_ref[...] + mu * pert_ref[...]
    m = jnp.max(logits, axis=-1, keepdims=True)
    t = logits - m
    s = jnp.sum(jnp.exp(t), axis=-1, keepdims=True)
    c = logits.shape[1]
    col = lax.broadcasted_iota(jnp.int32, (1, c), 1)
    tgt_m = jnp.sum(jnp.where(col == y_ref[...], t, 0.0), axis=-1, keepdims=True)
    loss_ref[...] = jnp.log(s) - tgt_m


def kernel(output, perturbation, y):
    n, c = output.shape
    tile_n = n
    for cand in (512, 256, 128, 64, 32, 16, 8):
        if n % cand == 0:
            tile_n = cand
            break
    y2 = y.astype(jnp.int32).reshape(n, 1)
    loss = pl.pallas_call(
        functools.partial(_loss_kernel, mu=_MU),
        grid=(n // tile_n,),
        in_specs=[
            pl.BlockSpec((tile_n, c), lambda i: (i, 0)),
            pl.BlockSpec((tile_n, c), lambda i: (i, 0)),
            pl.BlockSpec((tile_n, 1), lambda i: (i, 0)),
        ],
        out_specs=pl.BlockSpec((tile_n, 1), lambda i: (i, 0)),
        out_shape=jax.ShapeDtypeStruct((n, 1), jnp.float32),
        compiler_params=pltpu.CompilerParams(
            dimension_semantics=("parallel",),
            vmem_limit_bytes=48 * 1024 * 1024),
    )(output, perturbation, y2)
    return jnp.mean(loss[:, 0])


# speedup vs baseline: 1.5081x; 1.0191x over previous
import functools

import jax
import jax.numpy as jnp
from jax import lax
from jax.experimental import pallas as pl
from jax.experimental.pallas import tpu as pltpu

_MU = 0.5


def _loss_kernel(out_ref, pert_ref, y_ref, loss_ref, *, mu):
    logits = out_ref[...] + mu * pert_ref[...]
    m = jnp.max(logits, axis=-1, keepdims=True)
    t = logits - m
    s = jnp.sum(jnp.exp(t), axis=-1, keepdims=True)
    c = logits.shape[1]
    col = lax.broadcasted_iota(jnp.int32, (1, c), 1)
    tgt_m = jnp.sum(jnp.where(col == y_ref[...], t, 0.0), axis=-1, keepdims=True)
    loss_ref[...] = jnp.log(s) - tgt_m


def kernel(output, perturbation, y):
    n, c = output.shape
    tile_n = n
    for cand in (1024, 512, 256, 128, 64, 32, 16, 8):
        if n % cand == 0:
            tile_n = cand
            break
    y2 = y.astype(jnp.int32).reshape(n, 1)
    loss = pl.pallas_call(
        functools.partial(_loss_kernel, mu=_MU),
        grid=(n // tile_n,),
        in_specs=[
            pl.BlockSpec((tile_n, c), lambda i: (i, 0)),
            pl.BlockSpec((tile_n, c), lambda i: (i, 0)),
            pl.BlockSpec((tile_n, 1), lambda i: (i, 0)),
        ],
        out_specs=pl.BlockSpec((tile_n, 1), lambda i: (i, 0)),
        out_shape=jax.ShapeDtypeStruct((n, 1), jnp.float32),
        compiler_params=pltpu.CompilerParams(
            dimension_semantics=("parallel",),
            vmem_limit_bytes=48 * 1024 * 1024),
    )(output, perturbation, y2)
    return jnp.mean(loss[:, 0])
